# baseline (speedup 1.0000x reference)
import jax
import jax.numpy as jnp
from jax import lax
from jax.experimental import pallas as pl
from jax.experimental.pallas import tpu as pltpu

B, S, D = 32, 8, 768
E = 16
H = 2 * D
N = B * S
TB = 32
NBLK = 24

_F32 = jnp.float32
_I32 = jnp.int32


def _dot(a, b, dims):
    return lax.dot_general(a, b, (dims, ((), ())), preferred_element_type=_F32)


def _router_kernel(x_ref, wg_ref, pos_ref, be_ref, base_ref, limit_ref):
    x = x_ref[...]
    wg = wg_ref[...]
    gate = _dot(x, wg, ((1,), (1,)))

    e_iota = lax.broadcasted_iota(_I32, (N, E), 1)
    mx = jnp.max(gate, axis=1, keepdims=True)
    eid = jnp.min(jnp.where(gate == mx, e_iota, E), axis=1, keepdims=True)
    oh = (e_iota == eid).astype(_F32)

    cnt = jnp.sum(oh, axis=0, keepdims=True)
    lt16 = (lax.broadcasted_iota(_I32, (E, E), 0)
            < lax.broadcasted_iota(_I32, (E, E), 1)).astype(_F32)
    off = _dot(cnt, lt16, ((1,), (0,)))
    le256 = (lax.broadcasted_iota(_I32, (N, N), 1)
             <= lax.broadcasted_iota(_I32, (N, N), 0)).astype(_F32)
    ranks = _dot(le256, oh, ((1,), (0,)))
    rank = jnp.sum(ranks * oh, axis=1, keepdims=True)
    off_tok = jnp.sum(off * oh, axis=1, keepdims=True)
    pos = off_tok + rank - 1.0
    pos_ref[...] = pos.astype(_I32)

    cnt_i = cnt.astype(_I32)
    nblk = (cnt_i + (TB - 1)) // TB
    blkstart = _dot(nblk.astype(_F32), lt16, ((1,), (0,)))
    b_iota = lax.broadcasted_iota(_I32, (NBLK, E), 0).astype(_F32)
    e_iota2 = lax.broadcasted_iota(_I32, (NBLK, E), 1)
    cond = (blkstart <= b_iota) & (nblk > 0)
    bev = jnp.max(jnp.where(cond, e_iota2, -1), axis=1, keepdims=True)
    ohb = (e_iota2 == bev).astype(_F32)
    bs_b = jnp.sum(blkstart * ohb, axis=1, keepdims=True)
    off_b = jnp.sum(off * ohb, axis=1, keepdims=True)
    cnt_b = jnp.sum(cnt * ohb, axis=1, keepdims=True)
    lb = lax.broadcasted_iota(_I32, (NBLK, 1), 0).astype(_F32) - bs_b
    base = off_b + lb * TB
    limit = off_b + cnt_b
    be_ref[...] = bev.astype(_I32)
    base_ref[...] = base.astype(_I32)
    limit_ref[...] = limit.astype(_I32)


NH = 4
HC = H // NH


def _expert_kernel(be_s, base_s, limit_s, x_ref, pos_ref,
                   w1_ref, w2_ref, w3_ref, out_ref,
                   p2_scr, xblk_scr, yacc_scr):
    b = pl.program_id(0)
    hc = pl.program_id(1)

    @pl.when(hc == 0)
    def _():
        base = base_s[b]
        limit = limit_s[b]
        posv = pos_ref[...]
        r_iota = lax.broadcasted_iota(_I32, (N, TB), 1)
        p2 = ((posv - base == r_iota) & (posv < limit)).astype(_F32)
        p2_scr[...] = p2
        xblk_scr[...] = _dot(p2, x_ref[...], ((0,), (0,)))

    xblk = xblk_scr[...]
    h = _dot(xblk, w1_ref[0], ((1,), (1,)))
    v = _dot(xblk, w2_ref[0], ((1,), (1,)))
    act = h * jax.nn.sigmoid(h) * v
    ypart = _dot(act, w3_ref[0], ((1,), (1,)))

    @pl.when(hc == 0)
    def _():
        yacc_scr[...] = ypart

    @pl.when(hc > 0)
    def _():
        yacc_scr[...] += ypart

    @pl.when((b == 0) & (hc == 0))
    def _():
        out_ref[...] = jnp.zeros_like(out_ref)

    @pl.when(hc == NH - 1)
    def _():
        out_ref[...] += _dot(p2_scr[...], yacc_scr[...], ((1,), (0,)))


def kernel(x, Wg, W1, W2, W3):
    x2 = x.reshape(N, D)
    pos, be, base, limit = pl.pallas_call(
        _router_kernel,
        out_shape=[
            jax.ShapeDtypeStruct((N, 1), _I32),
            jax.ShapeDtypeStruct((NBLK, 1), _I32),
            jax.ShapeDtypeStruct((NBLK, 1), _I32),
            jax.ShapeDtypeStruct((NBLK, 1), _I32),
        ],
    )(x2, Wg)

    grid_spec = pltpu.PrefetchScalarGridSpec(
        num_scalar_prefetch=3,
        grid=(NBLK, NH),
        in_specs=[
            pl.BlockSpec((N, D), lambda b, hc, be_r, ba_r, li_r: (0, 0)),
            pl.BlockSpec((N, 1), lambda b, hc, be_r, ba_r, li_r: (0, 0)),
            pl.BlockSpec((1, HC, D),
                         lambda b, hc, be_r, ba_r, li_r: (be_r[b], hc, 0)),
            pl.BlockSpec((1, HC, D),
                         lambda b, hc, be_r, ba_r, li_r: (be_r[b], hc, 0)),
            pl.BlockSpec((1, D, HC),
                         lambda b, hc, be_r, ba_r, li_r: (be_r[b], 0, hc)),
        ],
        out_specs=pl.BlockSpec((N, D), lambda b, hc, be_r, ba_r, li_r: (0, 0)),
        scratch_shapes=[
            pltpu.VMEM((N, TB), _F32),
            pltpu.VMEM((TB, D), _F32),
            pltpu.VMEM((TB, D), _F32),
        ],
    )
    out = pl.pallas_call(
        _expert_kernel,
        grid_spec=grid_spec,
        out_shape=jax.ShapeDtypeStruct((N, D), _F32),
        compiler_params=pltpu.CompilerParams(
            dimension_semantics=("arbitrary", "arbitrary"),
        ),
    )(be.reshape(NBLK), base.reshape(NBLK), limit.reshape(NBLK),
      x2, pos, W1, W2, W3)
    return out.reshape(B, S, D)


# speedup vs baseline: 1.9513x; 1.9513x over previous
import jax
import jax.numpy as jnp
from jax import lax
from jax.experimental import pallas as pl
from jax.experimental.pallas import tpu as pltpu

B, S, D = 32, 8, 768
E = 16
H = 2 * D
N = B * S
TB = 32

_F32 = jnp.float32
_I32 = jnp.int32


def _dot(a, b, dims):
    return lax.dot_general(a, b, (dims, ((), ())), preferred_element_type=_F32)


def _router_kernel(x_ref, wg_ref, pos_ref, cnt_ref, off_ref):
    x = x_ref[...]
    wg = wg_ref[...]
    gate = _dot(x, wg, ((1,), (1,)))

    e_iota = lax.broadcasted_iota(_I32, (N, E), 1)
    mx = jnp.max(gate, axis=1, keepdims=True)
    eid = jnp.min(jnp.where(gate == mx, e_iota, E), axis=1, keepdims=True)
    oh = (e_iota == eid).astype(_F32)

    cnt = jnp.sum(oh, axis=0, keepdims=True)
    lt16 = (lax.broadcasted_iota(_I32, (E, E), 0)
            < lax.broadcasted_iota(_I32, (E, E), 1)).astype(_F32)
    off = _dot(cnt, lt16, ((1,), (0,)))
    le256 = (lax.broadcasted_iota(_I32, (N, N), 1)
             <= lax.broadcasted_iota(_I32, (N, N), 0)).astype(_F32)
    ranks = _dot(le256, oh, ((1,), (0,)))
    rank = jnp.sum(ranks * oh, axis=1, keepdims=True)
    off_tok = jnp.sum(off * oh, axis=1, keepdims=True)
    pos = off_tok + rank - 1.0
    pos_ref[...] = pos.astype(_I32)
    cnt_ref[...] = cnt.astype(_I32)
    off_ref[...] = off.astype(_I32)


def _expert_kernel(cnt_s, off_s, x_ref, pos_ref, w1_hbm, w2_hbm, w3_hbm,
                   out_ref, wb1, wb2, wb3, sem):
    posv = pos_ref[...]
    xall = x_ref[...]
    r_iota = lax.broadcasted_iota(_I32, (N, TB), 1)
    out_ref[...] = jnp.zeros_like(out_ref)

    def issue(e, slot):
        pltpu.make_async_copy(w1_hbm.at[e], wb1.at[slot], sem.at[slot, 0]).start()
        pltpu.make_async_copy(w2_hbm.at[e], wb2.at[slot], sem.at[slot, 1]).start()
        pltpu.make_async_copy(w3_hbm.at[e], wb3.at[slot], sem.at[slot, 2]).start()

    def wait(e, slot):
        pltpu.make_async_copy(w1_hbm.at[e], wb1.at[slot], sem.at[slot, 0]).wait()
        pltpu.make_async_copy(w2_hbm.at[e], wb2.at[slot], sem.at[slot, 1]).wait()
        pltpu.make_async_copy(w3_hbm.at[e], wb3.at[slot], sem.at[slot, 2]).wait()

    issue(0, 0)
    for e in range(E):
        slot = e % 2
        if e + 1 < E:
            issue(e + 1, (e + 1) % 2)
        wait(e, slot)
        w1 = wb1[slot]
        w2 = wb2[slot]
        w3 = wb3[slot]
        off_e = off_s[e]
        cnt_e = cnt_s[e]
        nblk_e = (cnt_e + (TB - 1)) // TB
        limit = off_e + cnt_e

        def body(j, _):
            base = off_e + j * TB
            p2 = ((posv - base == r_iota) & (posv < limit)).astype(_F32)
            xblk = _dot(p2, xall, ((0,), (0,)))
            h = _dot(xblk, w1, ((1,), (1,)))
            v = _dot(xblk, w2, ((1,), (1,)))
            act = h * jax.nn.sigmoid(h) * v
            y = _dot(act, w3, ((1,), (1,)))
            out_ref[...] += _dot(p2, y, ((1,), (0,)))
            return 0

        lax.fori_loop(0, nblk_e, body, 0)


def kernel(x, Wg, W1, W2, W3):
    x2 = x.reshape(N, D)
    pos, cnt, off = pl.pallas_call(
        _router_kernel,
        out_shape=[
            jax.ShapeDtypeStruct((N, 1), _I32),
            jax.ShapeDtypeStruct((1, E), _I32),
            jax.ShapeDtypeStruct((1, E), _I32),
        ],
    )(x2, Wg)

    out = pl.pallas_call(
        _expert_kernel,
        in_specs=[
            pl.BlockSpec(memory_space=pltpu.SMEM),
            pl.BlockSpec(memory_space=pltpu.SMEM),
            pl.BlockSpec(memory_space=pltpu.VMEM),
            pl.BlockSpec(memory_space=pltpu.VMEM),
            pl.BlockSpec(memory_space=pl.ANY),
            pl.BlockSpec(memory_space=pl.ANY),
            pl.BlockSpec(memory_space=pl.ANY),
        ],
        out_shape=jax.ShapeDtypeStruct((N, D), _F32),
        scratch_shapes=[
            pltpu.VMEM((2, H, D), _F32),
            pltpu.VMEM((2, H, D), _F32),
            pltpu.VMEM((2, D, H), _F32),
            pltpu.SemaphoreType.DMA((2, 3)),
        ],
        compiler_params=pltpu.CompilerParams(
            vmem_limit_bytes=100 * 1024 * 1024,
        ),
    )(cnt.reshape(E), off.reshape(E), x2, pos, W1, W2, W3)
    return out.reshape(B, S, D)


# speedup vs baseline: 2.0274x; 1.0390x over previous
import jax
import jax.numpy as jnp
from jax import lax
from jax.experimental import pallas as pl
from jax.experimental.pallas import tpu as pltpu

B, S, D = 32, 8, 768
E = 16
H = 2 * D
N = B * S
TB = 32

_F32 = jnp.float32
_I32 = jnp.int32


def _dot(a, b, dims):
    return lax.dot_general(a, b, (dims, ((), ())), preferred_element_type=_F32)


def _router_kernel(x_ref, wg_ref, pos_ref, cnt_ref, off_ref):
    x = x_ref[...]
    wg = wg_ref[...]
    gate = _dot(x, wg, ((1,), (1,)))

    e_iota = lax.broadcasted_iota(_I32, (N, E), 1)
    mx = jnp.max(gate, axis=1, keepdims=True)
    eid = jnp.min(jnp.where(gate == mx, e_iota, E), axis=1, keepdims=True)
    oh = (e_iota == eid).astype(_F32)

    cnt = jnp.sum(oh, axis=0, keepdims=True)
    lt16 = (lax.broadcasted_iota(_I32, (E, E), 0)
            < lax.broadcasted_iota(_I32, (E, E), 1)).astype(_F32)
    off = _dot(cnt, lt16, ((1,), (0,)))
    le256 = (lax.broadcasted_iota(_I32, (N, N), 1)
             <= lax.broadcasted_iota(_I32, (N, N), 0)).astype(_F32)
    ranks = _dot(le256, oh, ((1,), (0,)))
    rank = jnp.sum(ranks * oh, axis=1, keepdims=True)
    off_tok = jnp.sum(off * oh, axis=1, keepdims=True)
    pos = off_tok + rank - 1.0
    pos_ref[...] = pos.astype(_I32)
    cnt_ref[...] = cnt.astype(_I32)
    off_ref[...] = off.astype(_I32)


def _expert_kernel(cnt_s, off_s, x_ref, pos_ref, w1_hbm, w2_hbm, w3_hbm,
                   out_ref, wb1, wb2, wb3, sem):
    posv = pos_ref[...]
    xall = x_ref[...]
    r_iota = lax.broadcasted_iota(_I32, (N, TB), 1)
    out_ref[...] = jnp.zeros_like(out_ref)

    def issue(e, slot):
        pltpu.make_async_copy(w1_hbm.at[e], wb1.at[slot], sem.at[slot, 0]).start()
        pltpu.make_async_copy(w2_hbm.at[e], wb2.at[slot], sem.at[slot, 1]).start()
        pltpu.make_async_copy(w3_hbm.at[e], wb3.at[slot], sem.at[slot, 2]).start()

    def wait(e, slot):
        pltpu.make_async_copy(w1_hbm.at[e], wb1.at[slot], sem.at[slot, 0]).wait()
        pltpu.make_async_copy(w2_hbm.at[e], wb2.at[slot], sem.at[slot, 1]).wait()
        pltpu.make_async_copy(w3_hbm.at[e], wb3.at[slot], sem.at[slot, 2]).wait()

    issue(0, 0)
    issue(1, 1)
    for e in range(E):
        slot = e % 3
        if e + 2 < E:
            issue(e + 2, (e + 2) % 3)
        wait(e, slot)
        w1 = wb1[slot]
        w2 = wb2[slot]
        w3 = wb3[slot]
        off_e = off_s[e]
        cnt_e = cnt_s[e]
        nblk_e = (cnt_e + (TB - 1)) // TB
        limit = off_e + cnt_e

        def body(j, _):
            base = off_e + j * TB
            p2 = ((posv - base == r_iota) & (posv < limit)).astype(_F32)
            xblk = _dot(p2, xall, ((0,), (0,)))
            h = _dot(xblk, w1, ((1,), (1,)))
            v = _dot(xblk, w2, ((1,), (1,)))
            act = h * jax.nn.sigmoid(h) * v
            y = _dot(act, w3, ((1,), (1,)))
            out_ref[...] += _dot(p2, y, ((1,), (0,)))
            return 0

        lax.fori_loop(0, nblk_e, body, 0)


def kernel(x, Wg, W1, W2, W3):
    x2 = x.reshape(N, D)
    pos, cnt, off = pl.pallas_call(
        _router_kernel,
        out_shape=[
            jax.ShapeDtypeStruct((N, 1), _I32),
            jax.ShapeDtypeStruct((1, E), _I32),
            jax.ShapeDtypeStruct((1, E), _I32),
        ],
    )(x2, Wg)

    out = pl.pallas_call(
        _expert_kernel,
        in_specs=[
            pl.BlockSpec(memory_space=pltpu.SMEM),
            pl.BlockSpec(memory_space=pltpu.SMEM),
            pl.BlockSpec(memory_space=pltpu.VMEM),
            pl.BlockSpec(memory_space=pltpu.VMEM),
            pl.BlockSpec(memory_space=pl.ANY),
            pl.BlockSpec(memory_space=pl.ANY),
            pl.BlockSpec(memory_space=pl.ANY),
        ],
        out_shape=jax.ShapeDtypeStruct((N, D), _F32),
        scratch_shapes=[
            pltpu.VMEM((3, H, D), _F32),
            pltpu.VMEM((3, H, D), _F32),
            pltpu.VMEM((3, D, H), _F32),
            pltpu.SemaphoreType.DMA((3, 3)),
        ],
        compiler_params=pltpu.CompilerParams(
            vmem_limit_bytes=100 * 1024 * 1024,
        ),
    )(cnt.reshape(E), off.reshape(E), x2, pos, W1, W2, W3)
    return out.reshape(B, S, D)


# speedup vs baseline: 2.0486x; 1.0105x over previous
import jax
import jax.numpy as jnp
from jax import lax
from jax.experimental import pallas as pl
from jax.experimental.pallas import tpu as pltpu

B, S, D = 32, 8, 768
E = 16
H = 2 * D
N = B * S
TB = 32

_F32 = jnp.float32
_I32 = jnp.int32


def _dot(a, b, dims):
    return lax.dot_general(a, b, (dims, ((), ())), preferred_element_type=_F32)


def _router_kernel(x_ref, wg_ref, pos_ref, cnt_ref, off_ref):
    x = x_ref[...]
    wg = wg_ref[...]
    gate = _dot(x, wg, ((1,), (1,)))

    e_iota = lax.broadcasted_iota(_I32, (N, E), 1)
    mx = jnp.max(gate, axis=1, keepdims=True)
    eid = jnp.min(jnp.where(gate == mx, e_iota, E), axis=1, keepdims=True)
    oh = (e_iota == eid).astype(_F32)

    cnt = jnp.sum(oh, axis=0, keepdims=True)
    lt16 = (lax.broadcasted_iota(_I32, (E, E), 0)
            < lax.broadcasted_iota(_I32, (E, E), 1)).astype(_F32)
    off = _dot(cnt, lt16, ((1,), (0,)))
    le256 = (lax.broadcasted_iota(_I32, (N, N), 1)
             <= lax.broadcasted_iota(_I32, (N, N), 0)).astype(_F32)
    ranks = _dot(le256, oh, ((1,), (0,)))
    rank = jnp.sum(ranks * oh, axis=1, keepdims=True)
    off_tok = jnp.sum(off * oh, axis=1, keepdims=True)
    pos = off_tok + rank - 1.0
    pos_ref[...] = pos.astype(_I32)
    cnt_ref[...] = cnt.astype(_I32)
    off_ref[...] = off.astype(_I32)


def _expert_kernel(cnt_s, off_s, x_ref, pos_ref, w1_hbm, w2_hbm, w3_hbm,
                   out_ref, wb1, wb2, wb3, sem):
    posv = pos_ref[...]
    xall = x_ref[...]
    r_iota = lax.broadcasted_iota(_I32, (N, TB), 1)
    out_ref[...] = jnp.zeros_like(out_ref)

    def _copies(e, slot):
        hh = H // 2
        return [
            pltpu.make_async_copy(w1_hbm.at[e, pl.ds(0, hh)],
                                  wb1.at[slot, pl.ds(0, hh)], sem.at[slot, 0]),
            pltpu.make_async_copy(w1_hbm.at[e, pl.ds(hh, hh)],
                                  wb1.at[slot, pl.ds(hh, hh)], sem.at[slot, 1]),
            pltpu.make_async_copy(w2_hbm.at[e, pl.ds(0, hh)],
                                  wb2.at[slot, pl.ds(0, hh)], sem.at[slot, 2]),
            pltpu.make_async_copy(w2_hbm.at[e, pl.ds(hh, hh)],
                                  wb2.at[slot, pl.ds(hh, hh)], sem.at[slot, 3]),
            pltpu.make_async_copy(w3_hbm.at[e, pl.ds(0, D // 2)],
                                  wb3.at[slot, pl.ds(0, D // 2)], sem.at[slot, 4]),
            pltpu.make_async_copy(w3_hbm.at[e, pl.ds(D // 2, D // 2)],
                                  wb3.at[slot, pl.ds(D // 2, D // 2)], sem.at[slot, 5]),
        ]

    def issue(e, slot):
        for c in _copies(e, slot):
            c.start()

    def wait(e, slot):
        for c in _copies(e, slot):
            c.wait()

    issue(0, 0)
    issue(1, 1)
    for e in range(E):
        slot = e % 3
        if e + 2 < E:
            issue(e + 2, (e + 2) % 3)
        wait(e, slot)
        w1 = wb1[slot]
        w2 = wb2[slot]
        w3 = wb3[slot]
        off_e = off_s[e]
        cnt_e = cnt_s[e]
        nblk_e = (cnt_e + (TB - 1)) // TB
        limit = off_e + cnt_e

        def body(j, _):
            base = off_e + j * TB
            p2 = ((posv - base == r_iota) & (posv < limit)).astype(_F32)
            xblk = _dot(p2, xall, ((0,), (0,)))
            h = _dot(xblk, w1, ((1,), (1,)))
            v = _dot(xblk, w2, ((1,), (1,)))
            act = h * jax.nn.sigmoid(h) * v
            y = _dot(act, w3, ((1,), (1,)))
            out_ref[...] += _dot(p2, y, ((1,), (0,)))
            return 0

        lax.fori_loop(0, nblk_e, body, 0)


def kernel(x, Wg, W1, W2, W3):
    x2 = x.reshape(N, D)
    pos, cnt, off = pl.pallas_call(
        _router_kernel,
        out_shape=[
            jax.ShapeDtypeStruct((N, 1), _I32),
            jax.ShapeDtypeStruct((1, E), _I32),
            jax.ShapeDtypeStruct((1, E), _I32),
        ],
    )(x2, Wg)

    out = pl.pallas_call(
        _expert_kernel,
        in_specs=[
            pl.BlockSpec(memory_space=pltpu.SMEM),
            pl.BlockSpec(memory_space=pltpu.SMEM),
            pl.BlockSpec(memory_space=pltpu.VMEM),
            pl.BlockSpec(memory_space=pltpu.VMEM),
            pl.BlockSpec(memory_space=pl.ANY),
            pl.BlockSpec(memory_space=pl.ANY),
            pl.BlockSpec(memory_space=pl.ANY),
        ],
        out_shape=jax.ShapeDtypeStruct((N, D), _F32),
        scratch_shapes=[
            pltpu.VMEM((3, H, D), _F32),
            pltpu.VMEM((3, H, D), _F32),
            pltpu.VMEM((3, D, H), _F32),
            pltpu.SemaphoreType.DMA((3, 6)),
        ],
        compiler_params=pltpu.CompilerParams(
            vmem_limit_bytes=100 * 1024 * 1024,
        ),
    )(cnt.reshape(E), off.reshape(E), x2, pos, W1, W2, W3)
    return out.reshape(B, S, D)


# speedup vs baseline: 2.0948x; 1.0226x over previous
import jax
import jax.numpy as jnp
from jax import lax
from jax.experimental import pallas as pl
from jax.experimental.pallas import tpu as pltpu

B, S, D = 32, 8, 768
E = 16
H = 2 * D
N = B * S
TB = 32

_F32 = jnp.float32
_I32 = jnp.int32


def _dot(a, b, dims):
    return lax.dot_general(a, b, (dims, ((), ())), preferred_element_type=_F32)


def _moe_kernel(x_ref, wg_ref, w1_hbm, w2_hbm, w3_hbm, out_ref,
                wb1, wb2, wb3, sem, meta_vm, meta_sm, sem2):
    def _copies(e, slot):
        hh = H // 2
        hd = D // 2
        return [
            pltpu.make_async_copy(w1_hbm.at[e, pl.ds(0, hh)],
                                  wb1.at[slot, pl.ds(0, hh)], sem.at[slot, 0]),
            pltpu.make_async_copy(w1_hbm.at[e, pl.ds(hh, hh)],
                                  wb1.at[slot, pl.ds(hh, hh)], sem.at[slot, 1]),
            pltpu.make_async_copy(w2_hbm.at[e, pl.ds(0, hh)],
                                  wb2.at[slot, pl.ds(0, hh)], sem.at[slot, 2]),
            pltpu.make_async_copy(w2_hbm.at[e, pl.ds(hh, hh)],
                                  wb2.at[slot, pl.ds(hh, hh)], sem.at[slot, 3]),
            pltpu.make_async_copy(w3_hbm.at[e, pl.ds(0, hd)],
                                  wb3.at[slot, pl.ds(0, hd)], sem.at[slot, 4]),
            pltpu.make_async_copy(w3_hbm.at[e, pl.ds(hd, hd)],
                                  wb3.at[slot, pl.ds(hd, hd)], sem.at[slot, 5]),
        ]

    def issue(e, slot):
        for c in _copies(e, slot):
            c.start()

    def wait(e, slot):
        for c in _copies(e, slot):
            c.wait()

    issue(0, 0)
    issue(1, 1)

    x = x_ref[...]
    wg = wg_ref[...]
    gate = _dot(x, wg, ((1,), (1,)))

    e_iota = lax.broadcasted_iota(_I32, (N, E), 1)
    mx = jnp.max(gate, axis=1, keepdims=True)
    eid = jnp.min(jnp.where(gate == mx, e_iota, E), axis=1, keepdims=True)
    oh = (e_iota == eid).astype(_F32)

    cnt = jnp.sum(oh, axis=0, keepdims=True)
    lt16 = (lax.broadcasted_iota(_I32, (E, E), 0)
            < lax.broadcasted_iota(_I32, (E, E), 1)).astype(_F32)
    off = _dot(cnt, lt16, ((1,), (0,)))
    le256 = (lax.broadcasted_iota(_I32, (N, N), 1)
             <= lax.broadcasted_iota(_I32, (N, N), 0)).astype(_F32)
    ranks = _dot(le256, oh, ((1,), (0,)))
    rank = jnp.sum(ranks * oh, axis=1, keepdims=True)
    off_tok = jnp.sum(off * oh, axis=1, keepdims=True)
    posv = (off_tok + rank - 1.0).astype(_I32)

    meta_vm[0:1, :] = cnt.astype(_I32)
    meta_vm[1:2, :] = off.astype(_I32)
    cp = pltpu.make_async_copy(meta_vm, meta_sm, sem2)
    cp.start()
    cp.wait()

    r_iota = lax.broadcasted_iota(_I32, (N, TB), 1)
    out_ref[...] = jnp.zeros_like(out_ref)

    for e in range(E):
        slot = e % 3
        if e + 2 < E:
            issue(e + 2, (e + 2) % 3)
        wait(e, slot)
        w1 = wb1[slot]
        w2 = wb2[slot]
        w3 = wb3[slot]
        cnt_e = meta_sm[0, e]
        off_e = meta_sm[1, e]
        nblk_e = (cnt_e + (TB - 1)) // TB
        limit = off_e + cnt_e

        def body(j, _):
            base = off_e + j * TB
            p2 = ((posv - base == r_iota) & (posv < limit)).astype(_F32)
            xblk = _dot(p2, x, ((0,), (0,)))
            h = _dot(xblk, w1, ((1,), (1,)))
            v = _dot(xblk, w2, ((1,), (1,)))
            act = h * jax.nn.sigmoid(h) * v
            y = _dot(act, w3, ((1,), (1,)))
            out_ref[...] += _dot(p2, y, ((1,), (0,)))
            return 0

        lax.fori_loop(0, nblk_e, body, 0)


def kernel(x, Wg, W1, W2, W3):
    x2 = x.reshape(N, D)
    out = pl.pallas_call(
        _moe_kernel,
        in_specs=[
            pl.BlockSpec(memory_space=pltpu.VMEM),
            pl.BlockSpec(memory_space=pltpu.VMEM),
            pl.BlockSpec(memory_space=pl.ANY),
            pl.BlockSpec(memory_space=pl.ANY),
            pl.BlockSpec(memory_space=pl.ANY),
        ],
        out_shape=jax.ShapeDtypeStruct((N, D), _F32),
        scratch_shapes=[
            pltpu.VMEM((3, H, D), _F32),
            pltpu.VMEM((3, H, D), _F32),
            pltpu.VMEM((3, D, H), _F32),
            pltpu.SemaphoreType.DMA((3, 6)),
            pltpu.VMEM((2, E), _I32),
            pltpu.SMEM((2, E), _I32),
            pltpu.SemaphoreType.DMA,
        ],
        compiler_params=pltpu.CompilerParams(
            vmem_limit_bytes=100 * 1024 * 1024,
        ),
    )(x2, Wg, W1, W2, W3)
    return out.reshape(B, S, D)
